# Initial kernel scaffold; baseline (speedup 1.0000x reference)
import functools
import math

import jax
import jax.numpy as jnp
from jax import lax
from jax.experimental import pallas as pl
from jax.experimental.pallas import tpu as pltpu
from jax.experimental.pallas import tpu_sc as plsc

TOPK = 32
RPAD = 64
NQ = 8192
NS = 8192
CHUNK = 128
NCHUNK = NS // CHUNK
NUM_WORKERS = 32
ROWS_PER_W = NQ // NUM_WORKERS


def _proj_body(x_ref, wt_ref, o_ref):
    o_ref[...] = jnp.dot(x_ref[...], wt_ref[...],
                         preferred_element_type=jnp.float32)


def _project(x, wt, br=1024):
    n = x.shape[0]
    d = x.shape[1]
    return pl.pallas_call(
        _proj_body,
        grid=(n // br,),
        in_specs=[pl.BlockSpec((br, d), lambda i: (i, 0)),
                  pl.BlockSpec((d, RPAD), lambda i: (0, 0))],
        out_specs=pl.BlockSpec((br, RPAD), lambda i: (i, 0)),
        out_shape=jax.ShapeDtypeStruct((n, RPAD), jnp.float32),
    )(x, wt)


def _score_body(scale, rq_ref, rkt_ref, mask_ref, s_ref, cm_ref):
    s = jnp.dot(rq_ref[...], rkt_ref[...],
                preferred_element_type=jnp.float32)
    s = s * scale + mask_ref[...]
    s_ref[...] = s
    br = s.shape[0]
    cm_ref[...] = jnp.max(s.reshape(br, NCHUNK, CHUNK), axis=2)


def _scores(rq, rkt, mask2d, scale, br=256):
    grid = NQ // br
    return pl.pallas_call(
        functools.partial(_score_body, scale),
        grid=(grid,),
        in_specs=[pl.BlockSpec((br, RPAD), lambda i: (i, 0)),
                  pl.BlockSpec((RPAD, NS), lambda i: (0, 0)),
                  pl.BlockSpec((1, NS), lambda i: (0, 0))],
        out_specs=[pl.BlockSpec((br, NS), lambda i: (i, 0)),
                   pl.BlockSpec((br, NCHUNK), lambda i: (i, 0))],
        out_shape=[jax.ShapeDtypeStruct((NQ, NS), jnp.float32),
                   jax.ShapeDtypeStruct((NQ, NCHUNK), jnp.float32)],
    )(rq, rkt, mask2d)


def _topk_body(scores_hbm, cmax_hbm, idx_hbm, val_hbm,
               row_v, m_v, idx_v, val_v):
    cc = lax.axis_index("c")
    ss = lax.axis_index("s")
    wid = ss * 2 + cc
    iota = lax.broadcasted_iota(jnp.int32, (16,), 0)
    NEG = jnp.float32(-jnp.inf)
    BIG = jnp.int32(1 << 30)

    def row_body(r, carry):
        row = wid * ROWS_PER_W + r
        pltpu.sync_copy(scores_hbm.at[row], row_v)
        pltpu.sync_copy(cmax_hbm.at[row], m_v)

        def it_body(i, carry2):
            m = [m_v[pl.ds(16 * j, 16)] for j in range(4)]
            mmv = jnp.maximum(jnp.maximum(m[0], m[1]),
                              jnp.maximum(m[2], m[3]))
            cmax = jnp.max(mmv)
            cand = None
            for j in range(4):
                cj = jnp.where(m[j] == cmax, iota + (16 * j), BIG)
                cand = cj if cand is None else jnp.minimum(cand, cj)
            cid = jnp.min(cand)
            start = cid * CHUNK
            xs, gs, best = [], [], None
            for j in range(8):
                x = row_v[pl.ds(start + 16 * j, 16)]
                g = start + (16 * j) + iota
                xs.append(x)
                gs.append(g)
                bj = jnp.where(x == cmax, g, BIG)
                best = bj if best is None else jnp.minimum(best, bj)
            p = jnp.min(best)
            nm = None
            for j in range(8):
                xm = jnp.where(gs[j] == p, NEG, xs[j])
                nm = xm if nm is None else jnp.maximum(nm, xm)
            row_v[p] = NEG
            m_v[cid] = jnp.max(nm)
            idx_v[i] = p
            val_v[i] = cmax
            return carry2

        lax.fori_loop(0, TOPK, it_body, 0)
        pltpu.sync_copy(idx_v, idx_hbm.at[row])
        pltpu.sync_copy(val_v, val_hbm.at[row])
        return carry

    lax.fori_loop(0, ROWS_PER_W, row_body, 0)


def _topk(scores, cmax):
    mesh = plsc.VectorSubcoreMesh(core_axis_name="c", subcore_axis_name="s")
    fn = pl.kernel(
        _topk_body,
        out_type=[jax.ShapeDtypeStruct((NQ, TOPK), jnp.int32),
                  jax.ShapeDtypeStruct((NQ, TOPK), jnp.float32)],
        mesh=mesh,
        scratch_types=[pltpu.VMEM((NS,), jnp.float32),
                       pltpu.VMEM((NCHUNK,), jnp.float32),
                       pltpu.VMEM((TOPK,), jnp.int32),
                       pltpu.VMEM((TOPK,), jnp.float32)],
    )
    return fn(scores, cmax)


def kernel(query, slot_keys, reliability_mask, W_router):
    b, s, d = query.shape
    r = W_router.shape[0]
    scale = 1.0 / math.sqrt(r)
    q2 = query.reshape(b * s, d)
    wt = jnp.zeros((d, RPAD), jnp.float32).at[:, :r].set(W_router.T)
    rq = _project(q2, wt)
    rk = _project(slot_keys, wt)
    scores, cmax = _scores(rq, rk.T, reliability_mask.reshape(1, NS), scale)
    idx, val = _topk(scores, cmax)
    return idx.reshape(b, s, TOPK), val.reshape(b, s, TOPK)


# speedup vs baseline: 15.5416x; 15.5416x over previous
import functools
import math

import jax
import jax.numpy as jnp
from jax import lax
from jax.experimental import pallas as pl
from jax.experimental.pallas import tpu as pltpu
from jax.experimental.pallas import tpu_sc as plsc

TOPK = 32
RPAD = 64
NQ = 8192
NS = 8192
CHUNK = 128
NCHUNK = NS // CHUNK
NUM_WORKERS = 32
ROWS_PER_W = NQ // NUM_WORKERS


def _proj_body(x_ref, wt_ref, o_ref):
    o_ref[...] = jnp.dot(x_ref[...], wt_ref[...],
                         preferred_element_type=jnp.float32)


def _project(x, wt, br=1024):
    n = x.shape[0]
    d = x.shape[1]
    return pl.pallas_call(
        _proj_body,
        grid=(n // br,),
        in_specs=[pl.BlockSpec((br, d), lambda i: (i, 0)),
                  pl.BlockSpec((d, RPAD), lambda i: (0, 0))],
        out_specs=pl.BlockSpec((br, RPAD), lambda i: (i, 0)),
        out_shape=jax.ShapeDtypeStruct((n, RPAD), jnp.float32),
    )(x, wt)


def _score_body(scale, rq_ref, rkt_ref, mask_ref, s_ref, cm_ref):
    s = jnp.dot(rq_ref[...], rkt_ref[...],
                preferred_element_type=jnp.float32)
    s = s * scale + mask_ref[...]
    s_ref[...] = s
    br = s.shape[0]
    cm_ref[...] = jnp.max(s.reshape(br, NCHUNK, CHUNK), axis=2)


def _scores(rq, rkt, mask2d, scale, br=256):
    grid = NQ // br
    return pl.pallas_call(
        functools.partial(_score_body, scale),
        grid=(grid,),
        in_specs=[pl.BlockSpec((br, RPAD), lambda i: (i, 0)),
                  pl.BlockSpec((RPAD, NS), lambda i: (0, 0)),
                  pl.BlockSpec((1, NS), lambda i: (0, 0))],
        out_specs=[pl.BlockSpec((br, NS), lambda i: (i, 0)),
                   pl.BlockSpec((br, NCHUNK), lambda i: (i, 0))],
        out_shape=[jax.ShapeDtypeStruct((NQ, NS), jnp.float32),
                   jax.ShapeDtypeStruct((NQ, NCHUNK), jnp.float32)],
    )(rq, rkt, mask2d)


def _topk_body(scores_hbm, cmax_hbm, idx_hbm, val_hbm,
               row_v, m_v, idx_v, val_v):
    cc = lax.axis_index("c")
    ss = lax.axis_index("s")
    wid = ss * 2 + cc
    iota = lax.broadcasted_iota(jnp.int32, (16,), 0)
    lane0 = iota == 0
    NEG = jnp.float32(-jnp.inf)
    BIG = jnp.int32(1 << 30)

    def _put(ref, pos, value):
        plsc.store_scatter(ref, [jnp.full((16,), pos, jnp.int32)],
                           jnp.full((16,), value, ref.dtype), mask=lane0)

    def _shuf(x, s):
        return x.at[iota ^ s].get(mode="promise_in_bounds")

    def _lanemax(x):
        for sh in (8, 4, 2, 1):
            x = jnp.maximum(x, _shuf(x, sh))
        return x

    def _lanemin(x):
        for sh in (8, 4, 2, 1):
            x = jnp.minimum(x, _shuf(x, sh))
        return x

    def row_body(r, carry):
        row = wid * ROWS_PER_W + r
        pltpu.sync_copy(scores_hbm.at[row], row_v)
        pltpu.sync_copy(cmax_hbm.at[row], m_v)

        def it_body(i, carry2):
            m = [m_v[pl.ds(16 * j, 16)] for j in range(4)]
            mmv = jnp.maximum(jnp.maximum(m[0], m[1]),
                              jnp.maximum(m[2], m[3]))
            cmax_v = _lanemax(mmv)
            cand = None
            for j in range(4):
                cj = jnp.where(m[j] == cmax_v, iota + (16 * j), BIG)
                cand = cj if cand is None else jnp.minimum(cand, cj)
            cid_v = _lanemin(cand)
            start = cid_v[0] * CHUNK
            xs, gs, best = [], [], None
            for j in range(8):
                x = row_v[pl.ds(start + 16 * j, 16)]
                g = (start + (16 * j)) + iota
                xs.append(x)
                gs.append(g)
                bj = jnp.where(x == cmax_v, g, BIG)
                best = bj if best is None else jnp.minimum(best, bj)
            p_v = _lanemin(best)
            nm = None
            for j in range(8):
                xm = jnp.where(gs[j] == p_v, NEG, xs[j])
                nm = xm if nm is None else jnp.maximum(nm, xm)
            _put(row_v, p_v[0], NEG)
            _put(m_v, cid_v[0], _lanemax(nm)[0])
            _put(idx_v, i, p_v[0])
            _put(val_v, i, cmax_v[0])
            return carry2

        lax.fori_loop(0, TOPK, it_body, 0)
        pltpu.sync_copy(idx_v, idx_hbm.at[row])
        pltpu.sync_copy(val_v, val_hbm.at[row])
        return carry

    lax.fori_loop(0, ROWS_PER_W, row_body, 0)


def _topk(scores, cmax):
    mesh = plsc.VectorSubcoreMesh(core_axis_name="c", subcore_axis_name="s")
    fn = pl.kernel(
        _topk_body,
        out_type=[jax.ShapeDtypeStruct((NQ, TOPK), jnp.int32),
                  jax.ShapeDtypeStruct((NQ, TOPK), jnp.float32)],
        mesh=mesh,
        compiler_params=pltpu.CompilerParams(needs_layout_passes=False),
        scratch_types=[pltpu.VMEM((NS,), jnp.float32),
                       pltpu.VMEM((NCHUNK,), jnp.float32),
                       pltpu.VMEM((TOPK,), jnp.int32),
                       pltpu.VMEM((TOPK,), jnp.float32)],
    )
    return fn(scores, cmax)


def kernel(query, slot_keys, reliability_mask, W_router):
    b, s, d = query.shape
    r = W_router.shape[0]
    scale = 1.0 / math.sqrt(r)
    q2 = query.reshape(b * s, d)
    wt = jnp.zeros((d, RPAD), jnp.float32).at[:, :r].set(W_router.T)
    rq = _project(q2, wt)
    rk = _project(slot_keys, wt)
    scores, cmax = _scores(rq, rk.T, reliability_mask.reshape(1, NS), scale)
    idx, val = _topk(scores, cmax)
    return idx.reshape(b, s, TOPK), val.reshape(b, s, TOPK)


# speedup vs baseline: 23.0047x; 1.4802x over previous
import functools
import math

import jax
import jax.numpy as jnp
from jax import lax
from jax.experimental import pallas as pl
from jax.experimental.pallas import tpu as pltpu
from jax.experimental.pallas import tpu_sc as plsc

TOPK = 32
RPAD = 64
NQ = 8192
NS = 8192
CHUNK = 128
NCHUNK = NS // CHUNK
NUM_WORKERS = 32
ROWS_PER_W = NQ // NUM_WORKERS


def _proj_body(x_ref, wt_ref, o_ref):
    o_ref[...] = jnp.dot(x_ref[...], wt_ref[...],
                         preferred_element_type=jnp.float32)


def _project(x, wt, br=1024):
    n = x.shape[0]
    d = x.shape[1]
    return pl.pallas_call(
        _proj_body,
        grid=(n // br,),
        in_specs=[pl.BlockSpec((br, d), lambda i: (i, 0)),
                  pl.BlockSpec((d, RPAD), lambda i: (0, 0))],
        out_specs=pl.BlockSpec((br, RPAD), lambda i: (i, 0)),
        out_shape=jax.ShapeDtypeStruct((n, RPAD), jnp.float32),
    )(x, wt)


def _score_body(scale, rq_ref, rkt_ref, mask_ref, s_ref, cm_ref):
    s = jnp.dot(rq_ref[...], rkt_ref[...],
                preferred_element_type=jnp.float32)
    s = s * scale + mask_ref[...]
    s_ref[...] = s
    br = s.shape[0]
    cm_ref[...] = jnp.max(s.reshape(br, NCHUNK, CHUNK), axis=2)


def _scores(rq, rkt, mask2d, scale, br=256):
    grid = NQ // br
    return pl.pallas_call(
        functools.partial(_score_body, scale),
        grid=(grid,),
        in_specs=[pl.BlockSpec((br, RPAD), lambda i: (i, 0)),
                  pl.BlockSpec((RPAD, NS), lambda i: (0, 0)),
                  pl.BlockSpec((1, NS), lambda i: (0, 0))],
        out_specs=[pl.BlockSpec((br, NS), lambda i: (i, 0)),
                   pl.BlockSpec((br, NCHUNK), lambda i: (i, 0))],
        out_shape=[jax.ShapeDtypeStruct((NQ, NS), jnp.float32),
                   jax.ShapeDtypeStruct((NQ, NCHUNK), jnp.float32)],
    )(rq, rkt, mask2d)


def _topk_body(scores_hbm, cmax_hbm, idx_hbm, val_hbm,
               row_a, row_b, m_all, idx_acc, val_acc, sem_a, sem_b):
    cc = lax.axis_index("c")
    ss = lax.axis_index("s")
    wid = ss * 2 + cc
    base = wid * ROWS_PER_W
    iota = lax.broadcasted_iota(jnp.int32, (16,), 0)
    lane0 = iota == 0
    NEG = jnp.float32(-jnp.inf)
    BIG = jnp.int32(1 << 30)

    def _put(ref, r, pos, value):
        plsc.store_scatter(ref,
                           [jnp.full((16,), r, jnp.int32),
                            jnp.full((16,), pos, jnp.int32)],
                           jnp.full((16,), value, ref.dtype), mask=lane0)

    def _shuf(x, s):
        return x.at[iota ^ s].get(mode="promise_in_bounds")

    def _lanemax(x):
        for sh in (8, 4, 2, 1):
            x = jnp.maximum(x, _shuf(x, sh))
        return x

    def _lanemin(x):
        for sh in (8, 4, 2, 1):
            x = jnp.minimum(x, _shuf(x, sh))
        return x

    pltpu.sync_copy(cmax_hbm.at[pl.ds(base, ROWS_PER_W)], m_all)
    pltpu.async_copy(scores_hbm.at[base], row_a, sem_a)

    def process(r, row_v):
        def it_body(i, carry2):
            m = [m_all[r, pl.ds(16 * j, 16)] for j in range(4)]
            mmv = jnp.maximum(jnp.maximum(m[0], m[1]),
                              jnp.maximum(m[2], m[3]))
            cmax_v = _lanemax(mmv)
            cand = None
            for j in range(4):
                cj = jnp.where(m[j] == cmax_v, iota + (16 * j), BIG)
                cand = cj if cand is None else jnp.minimum(cand, cj)
            cid_v = _lanemin(cand)
            start = cid_v[0] * CHUNK
            xs, gs, best = [], [], None
            for j in range(8):
                x = row_v[pl.ds(start + 16 * j, 16)]
                g = (start + (16 * j)) + iota
                xs.append(x)
                gs.append(g)
                bj = jnp.where(x == cmax_v, g, BIG)
                best = bj if best is None else jnp.minimum(best, bj)
            p_v = _lanemin(best)
            nm = None
            for j in range(8):
                xm = jnp.where(gs[j] == p_v, NEG, xs[j])
                nm = xm if nm is None else jnp.maximum(nm, xm)
            plsc.store_scatter(row_v, [p_v],
                               jnp.full((16,), NEG, jnp.float32), mask=lane0)
            _put(m_all, r, cid_v[0], _lanemax(nm)[0])
            _put(idx_acc, r, i, p_v[0])
            _put(val_acc, r, i, cmax_v[0])
            return carry2

        lax.fori_loop(0, TOPK, it_body, 0)

    def body2(r2, carry):
        r0 = 2 * r2
        r1 = r0 + 1
        pltpu.async_copy(scores_hbm.at[base + r1], row_b, sem_b)
        pltpu.make_async_copy(scores_hbm.at[base + r0], row_a, sem_a).wait()
        process(r0, row_a)

        @pl.when(r2 < ROWS_PER_W // 2 - 1)
        def _():
            pltpu.async_copy(scores_hbm.at[base + r0 + 2], row_a, sem_a)

        pltpu.make_async_copy(scores_hbm.at[base + r1], row_b, sem_b).wait()
        process(r1, row_b)
        return carry

    lax.fori_loop(0, ROWS_PER_W // 2, body2, 0)
    pltpu.sync_copy(idx_acc, idx_hbm.at[pl.ds(base, ROWS_PER_W)])
    pltpu.sync_copy(val_acc, val_hbm.at[pl.ds(base, ROWS_PER_W)])


def _topk(scores, cmax):
    mesh = plsc.VectorSubcoreMesh(core_axis_name="c", subcore_axis_name="s")
    fn = pl.kernel(
        _topk_body,
        out_type=[jax.ShapeDtypeStruct((NQ, TOPK), jnp.int32),
                  jax.ShapeDtypeStruct((NQ, TOPK), jnp.float32)],
        mesh=mesh,
        compiler_params=pltpu.CompilerParams(needs_layout_passes=False),
        scratch_types=[pltpu.VMEM((NS,), jnp.float32),
                       pltpu.VMEM((NS,), jnp.float32),
                       pltpu.VMEM((ROWS_PER_W, NCHUNK), jnp.float32),
                       pltpu.VMEM((ROWS_PER_W, TOPK), jnp.int32),
                       pltpu.VMEM((ROWS_PER_W, TOPK), jnp.float32),
                       pltpu.SemaphoreType.DMA,
                       pltpu.SemaphoreType.DMA],
    )
    return fn(scores, cmax)


def kernel(query, slot_keys, reliability_mask, W_router):
    b, s, d = query.shape
    r = W_router.shape[0]
    scale = 1.0 / math.sqrt(r)
    q2 = query.reshape(b * s, d)
    wt = jnp.zeros((d, RPAD), jnp.float32).at[:, :r].set(W_router.T)
    rq = _project(q2, wt)
    rk = _project(slot_keys, wt)
    scores, cmax = _scores(rq, rk.T, reliability_mask.reshape(1, NS), scale)
    idx, val = _topk(scores, cmax)
    return idx.reshape(b, s, TOPK), val.reshape(b, s, TOPK)


# speedup vs baseline: 28.8867x; 1.2557x over previous
import functools
import math

import jax
import jax.numpy as jnp
from jax import lax
from jax.experimental import pallas as pl
from jax.experimental.pallas import tpu as pltpu
from jax.experimental.pallas import tpu_sc as plsc

TOPK = 32
RPAD = 64
NQ = 8192
NS = 8192
CHUNK = 128
NCHUNK = NS // CHUNK
NUM_WORKERS = 32
ROWS_PER_W = NQ // NUM_WORKERS


def _proj_body(x_ref, wt_ref, o_ref):
    o_ref[...] = jnp.dot(x_ref[...], wt_ref[...],
                         preferred_element_type=jnp.float32)


def _project(x, wt, br=1024):
    n = x.shape[0]
    d = x.shape[1]
    return pl.pallas_call(
        _proj_body,
        grid=(n // br,),
        in_specs=[pl.BlockSpec((br, d), lambda i: (i, 0)),
                  pl.BlockSpec((d, RPAD), lambda i: (0, 0))],
        out_specs=pl.BlockSpec((br, RPAD), lambda i: (i, 0)),
        out_shape=jax.ShapeDtypeStruct((n, RPAD), jnp.float32),
    )(x, wt)


def _score_body(scale, rq_ref, rkt_ref, mask_ref, s_ref, cm_ref):
    s = jnp.dot(rq_ref[...], rkt_ref[...],
                preferred_element_type=jnp.float32)
    s = s * scale + mask_ref[...]
    s_ref[...] = s
    br = s.shape[0]
    cm_ref[...] = jnp.max(s.reshape(br, NCHUNK, CHUNK), axis=2)


def _scores(rq, rkt, mask2d, scale, br=256):
    grid = NQ // br
    return pl.pallas_call(
        functools.partial(_score_body, scale),
        grid=(grid,),
        in_specs=[pl.BlockSpec((br, RPAD), lambda i: (i, 0)),
                  pl.BlockSpec((RPAD, NS), lambda i: (0, 0)),
                  pl.BlockSpec((1, NS), lambda i: (0, 0))],
        out_specs=[pl.BlockSpec((br, NS), lambda i: (i, 0)),
                   pl.BlockSpec((br, NCHUNK), lambda i: (i, 0))],
        out_shape=[jax.ShapeDtypeStruct((NQ, NS), jnp.float32),
                   jax.ShapeDtypeStruct((NQ, NCHUNK), jnp.float32)],
    )(rq, rkt, mask2d)


def _topk_body(scores_hbm, cmax_hbm, idx_hbm, val_hbm,
               row_a, row_b, m_all, idx_acc, val_acc, sem_a, sem_b):
    cc = lax.axis_index("c")
    ss = lax.axis_index("s")
    wid = ss * 2 + cc
    base = wid * ROWS_PER_W
    iota = lax.broadcasted_iota(jnp.int32, (16,), 0)
    lane0 = iota == 0
    NEG = jnp.float32(-jnp.inf)
    BIG = jnp.int32(1 << 30)

    def _put(ref, r, pos, value):
        plsc.store_scatter(ref,
                           [jnp.full((16,), r, jnp.int32),
                            jnp.full((16,), pos, jnp.int32)],
                           jnp.full((16,), value, ref.dtype), mask=lane0)

    def _shuf(x, s):
        return x.at[iota ^ s].get(mode="promise_in_bounds")

    def _lanemax(x):
        for sh in (8, 4, 2, 1):
            x = jnp.maximum(x, _shuf(x, sh))
        return x

    def _lanemin(x):
        for sh in (8, 4, 2, 1):
            x = jnp.minimum(x, _shuf(x, sh))
        return x

    pltpu.sync_copy(cmax_hbm.at[pl.ds(base, ROWS_PER_W)], m_all)
    pltpu.async_copy(scores_hbm.at[base], row_a, sem_a)

    NEG_VEC = jnp.full((16,), NEG, jnp.float32)

    def process(r, row_v):
        def it_body(i, m):
            mmv = jnp.maximum(jnp.maximum(m[0], m[1]),
                              jnp.maximum(m[2], m[3]))
            cmax = jnp.max(mmv)
            cand = None
            for j in range(4):
                fj = plsc.all_reduce_ffs(m[j] == cmax)
                cj = jnp.where(fj < 16, fj + (16 * j), BIG)
                cand = cj if cand is None else jnp.minimum(cand, cj)
            cid_v = cand
            start = cid_v[0] * CHUNK
            xs, pos = [], None
            for j in range(8):
                x = row_v[pl.ds(start + 16 * j, 16)]
                xs.append(x)
                fj = plsc.all_reduce_ffs(x == cmax)
                pj = jnp.where(fj < 16, (start + 16 * j) + fj, BIG)
                pos = pj if pos is None else jnp.minimum(pos, pj)
            p_v = pos
            nm = None
            for j in range(8):
                d = p_v - (start + 16 * j)
                xm = jnp.where(iota == d, NEG, xs[j])
                nm = xm if nm is None else jnp.maximum(nm, xm)
            newmax = jnp.max(nm)
            plsc.store_scatter(row_v, [p_v], NEG_VEC, mask=lane0)
            _put(idx_acc, r, i, p_v[0])
            _put(val_acc, r, i, cmax)
            cdiv = cid_v >> 4
            cmod = cid_v & 15
            return tuple(
                jnp.where((iota == cmod) & (cdiv == j), newmax, m[j])
                for j in range(4))

        m0 = tuple(m_all[r, pl.ds(16 * j, 16)] for j in range(4))
        lax.fori_loop(0, TOPK, it_body, m0)

    def body2(r2, carry):
        r0 = 2 * r2
        r1 = r0 + 1
        pltpu.async_copy(scores_hbm.at[base + r1], row_b, sem_b)
        pltpu.make_async_copy(scores_hbm.at[base + r0], row_a, sem_a).wait()
        process(r0, row_a)

        @pl.when(r2 < ROWS_PER_W // 2 - 1)
        def _():
            pltpu.async_copy(scores_hbm.at[base + r0 + 2], row_a, sem_a)

        pltpu.make_async_copy(scores_hbm.at[base + r1], row_b, sem_b).wait()
        process(r1, row_b)
        return carry

    lax.fori_loop(0, ROWS_PER_W // 2, body2, 0)
    pltpu.sync_copy(idx_acc, idx_hbm.at[pl.ds(base, ROWS_PER_W)])
    pltpu.sync_copy(val_acc, val_hbm.at[pl.ds(base, ROWS_PER_W)])


def _topk(scores, cmax):
    mesh = plsc.VectorSubcoreMesh(core_axis_name="c", subcore_axis_name="s")
    fn = pl.kernel(
        _topk_body,
        out_type=[jax.ShapeDtypeStruct((NQ, TOPK), jnp.int32),
                  jax.ShapeDtypeStruct((NQ, TOPK), jnp.float32)],
        mesh=mesh,
        compiler_params=pltpu.CompilerParams(needs_layout_passes=False),
        scratch_types=[pltpu.VMEM((NS,), jnp.float32),
                       pltpu.VMEM((NS,), jnp.float32),
                       pltpu.VMEM((ROWS_PER_W, NCHUNK), jnp.float32),
                       pltpu.VMEM((ROWS_PER_W, TOPK), jnp.int32),
                       pltpu.VMEM((ROWS_PER_W, TOPK), jnp.float32),
                       pltpu.SemaphoreType.DMA,
                       pltpu.SemaphoreType.DMA],
    )
    return fn(scores, cmax)


def kernel(query, slot_keys, reliability_mask, W_router):
    b, s, d = query.shape
    r = W_router.shape[0]
    scale = 1.0 / math.sqrt(r)
    q2 = query.reshape(b * s, d)
    wt = jnp.zeros((d, RPAD), jnp.float32).at[:, :r].set(W_router.T)
    rq = _project(q2, wt)
    rk = _project(slot_keys, wt)
    scores, cmax = _scores(rq, rk.T, reliability_mask.reshape(1, NS), scale)
    idx, val = _topk(scores, cmax)
    return idx.reshape(b, s, TOPK), val.reshape(b, s, TOPK)


# speedup vs baseline: 40.0861x; 1.3877x over previous
import functools
import math

import jax
import jax.numpy as jnp
from jax import lax
from jax.experimental import pallas as pl
from jax.experimental.pallas import tpu as pltpu
from jax.experimental.pallas import tpu_sc as plsc

TOPK = 32
RPAD = 64
NQ = 8192
NS = 8192
CHUNK = 128
NCHUNK = NS // CHUNK
NUM_WORKERS = 32
ROWS_PER_W = NQ // NUM_WORKERS


def _proj_body(x_ref, wt_ref, o_ref):
    o_ref[...] = jnp.dot(x_ref[...], wt_ref[...],
                         preferred_element_type=jnp.float32)


def _project(x, wt, br=1024):
    n = x.shape[0]
    d = x.shape[1]
    return pl.pallas_call(
        _proj_body,
        grid=(n // br,),
        in_specs=[pl.BlockSpec((br, d), lambda i: (i, 0)),
                  pl.BlockSpec((d, RPAD), lambda i: (0, 0))],
        out_specs=pl.BlockSpec((br, RPAD), lambda i: (i, 0)),
        out_shape=jax.ShapeDtypeStruct((n, RPAD), jnp.float32),
    )(x, wt)


def _score_body(scale, rq_ref, rkt_ref, mask_ref, s_ref, cm_ref):
    s = jnp.dot(rq_ref[...], rkt_ref[...],
                preferred_element_type=jnp.float32)
    s = s * scale + mask_ref[...]
    s_ref[...] = s
    br = s.shape[0]
    cm_ref[...] = jnp.max(s.reshape(br, NCHUNK, CHUNK), axis=2)


def _scores(rq, rkt, mask2d, scale, br=256):
    grid = NQ // br
    return pl.pallas_call(
        functools.partial(_score_body, scale),
        grid=(grid,),
        in_specs=[pl.BlockSpec((br, RPAD), lambda i: (i, 0)),
                  pl.BlockSpec((RPAD, NS), lambda i: (0, 0)),
                  pl.BlockSpec((1, NS), lambda i: (0, 0))],
        out_specs=[pl.BlockSpec((br, NS), lambda i: (i, 0)),
                   pl.BlockSpec((br, NCHUNK), lambda i: (i, 0))],
        out_shape=[jax.ShapeDtypeStruct((NQ, NS), jnp.float32),
                   jax.ShapeDtypeStruct((NQ, NCHUNK), jnp.float32)],
    )(rq, rkt, mask2d)


def _topk_body(scores_hbm, cmax_hbm, idx_hbm, val_hbm,
               row_a, row_b, row_c, row_d, m_all, idx_acc, val_acc,
               sem_a, sem_b, sem_c, sem_d):
    cc = lax.axis_index("c")
    ss = lax.axis_index("s")
    wid = ss * 2 + cc
    base = wid * ROWS_PER_W
    iota = lax.broadcasted_iota(jnp.int32, (16,), 0)
    lane0 = iota == 0
    NEG = jnp.float32(-jnp.inf)
    BIG = jnp.int32(1 << 30)

    def _put(ref, r, pos, value):
        plsc.store_scatter(ref,
                           [jnp.full((16,), r, jnp.int32),
                            jnp.full((16,), pos, jnp.int32)],
                           jnp.full((16,), value, ref.dtype), mask=lane0)

    def _shuf(x, s):
        return x.at[iota ^ s].get(mode="promise_in_bounds")

    def _lanemax(x):
        for sh in (8, 4, 2, 1):
            x = jnp.maximum(x, _shuf(x, sh))
        return x

    def _lanemin(x):
        for sh in (8, 4, 2, 1):
            x = jnp.minimum(x, _shuf(x, sh))
        return x

    pltpu.sync_copy(cmax_hbm.at[pl.ds(base, ROWS_PER_W)], m_all)
    pltpu.async_copy(scores_hbm.at[base], row_a, sem_a)
    pltpu.async_copy(scores_hbm.at[base + 1], row_b, sem_b)

    NEG_VEC = jnp.full((16,), NEG, jnp.float32)

    def step(i, m, r, row_v):
        mmv = jnp.maximum(jnp.maximum(m[0], m[1]),
                          jnp.maximum(m[2], m[3]))
        cmax = jnp.max(mmv)
        cand = None
        for j in range(4):
            fj = plsc.all_reduce_ffs(m[j] == cmax)
            cj = jnp.where(fj < 16, fj + (16 * j), BIG)
            cand = cj if cand is None else jnp.minimum(cand, cj)
        cid_v = cand
        start = cid_v[0] * CHUNK
        xs, pos = [], None
        for j in range(8):
            x = row_v[pl.ds(start + 16 * j, 16)]
            xs.append(x)
            fj = plsc.all_reduce_ffs(x == cmax)
            pj = jnp.where(fj < 16, (start + 16 * j) + fj, BIG)
            pos = pj if pos is None else jnp.minimum(pos, pj)
        p_v = pos
        nm = None
        for j in range(8):
            d = p_v - (start + 16 * j)
            xm = jnp.where(iota == d, NEG, xs[j])
            nm = xm if nm is None else jnp.maximum(nm, xm)
        newmax = jnp.max(nm)
        plsc.store_scatter(row_v, [p_v], NEG_VEC, mask=lane0)
        _put(idx_acc, r, i, p_v[0])
        _put(val_acc, r, i, cmax)
        cdiv = cid_v >> 4
        cmod = cid_v & 15
        return tuple(
            jnp.where((iota == cmod) & (cdiv == j), newmax, m[j])
            for j in range(4))

    def process_pair(r, row_x, row_y):
        def it_body(i, m):
            ma = step(i, m[:4], r, row_x)
            mb = step(i, m[4:], r + 1, row_y)
            return ma + mb

        m0 = tuple(m_all[r, pl.ds(16 * j, 16)] for j in range(4))
        m1 = tuple(m_all[r + 1, pl.ds(16 * j, 16)] for j in range(4))
        lax.fori_loop(0, TOPK, it_body, m0 + m1)

    def body4(q, carry):
        r0 = 4 * q
        pltpu.async_copy(scores_hbm.at[base + r0 + 2], row_c, sem_c)
        pltpu.async_copy(scores_hbm.at[base + r0 + 3], row_d, sem_d)
        pltpu.make_async_copy(scores_hbm.at[base + r0], row_a, sem_a).wait()
        pltpu.make_async_copy(scores_hbm.at[base + r0 + 1], row_b, sem_b).wait()
        process_pair(r0, row_a, row_b)

        @pl.when(q < ROWS_PER_W // 4 - 1)
        def _():
            pltpu.async_copy(scores_hbm.at[base + r0 + 4], row_a, sem_a)
            pltpu.async_copy(scores_hbm.at[base + r0 + 5], row_b, sem_b)

        pltpu.make_async_copy(scores_hbm.at[base + r0 + 2], row_c, sem_c).wait()
        pltpu.make_async_copy(scores_hbm.at[base + r0 + 3], row_d, sem_d).wait()
        process_pair(r0 + 2, row_c, row_d)
        return carry

    lax.fori_loop(0, ROWS_PER_W // 4, body4, 0)
    pltpu.sync_copy(idx_acc, idx_hbm.at[pl.ds(base, ROWS_PER_W)])
    pltpu.sync_copy(val_acc, val_hbm.at[pl.ds(base, ROWS_PER_W)])


def _topk(scores, cmax):
    mesh = plsc.VectorSubcoreMesh(core_axis_name="c", subcore_axis_name="s")
    fn = pl.kernel(
        _topk_body,
        out_type=[jax.ShapeDtypeStruct((NQ, TOPK), jnp.int32),
                  jax.ShapeDtypeStruct((NQ, TOPK), jnp.float32)],
        mesh=mesh,
        compiler_params=pltpu.CompilerParams(needs_layout_passes=False),
        scratch_types=[pltpu.VMEM((NS,), jnp.float32),
                       pltpu.VMEM((NS,), jnp.float32),
                       pltpu.VMEM((NS,), jnp.float32),
                       pltpu.VMEM((NS,), jnp.float32),
                       pltpu.VMEM((ROWS_PER_W, NCHUNK), jnp.float32),
                       pltpu.VMEM((ROWS_PER_W, TOPK), jnp.int32),
                       pltpu.VMEM((ROWS_PER_W, TOPK), jnp.float32),
                       pltpu.SemaphoreType.DMA,
                       pltpu.SemaphoreType.DMA,
                       pltpu.SemaphoreType.DMA,
                       pltpu.SemaphoreType.DMA],
    )
    return fn(scores, cmax)


def kernel(query, slot_keys, reliability_mask, W_router):
    b, s, d = query.shape
    r = W_router.shape[0]
    scale = 1.0 / math.sqrt(r)
    q2 = query.reshape(b * s, d)
    wt = jnp.zeros((d, RPAD), jnp.float32).at[:, :r].set(W_router.T)
    rq = _project(q2, wt)
    rk = _project(slot_keys, wt)
    scores, cmax = _scores(rq, rk.T, reliability_mask.reshape(1, NS), scale)
    idx, val = _topk(scores, cmax)
    return idx.reshape(b, s, TOPK), val.reshape(b, s, TOPK)


# speedup vs baseline: 44.9535x; 1.1214x over previous
import functools
import math

import jax
import jax.numpy as jnp
from jax import lax
from jax.experimental import pallas as pl
from jax.experimental.pallas import tpu as pltpu
from jax.experimental.pallas import tpu_sc as plsc

TOPK = 32
RPAD = 64
NQ = 8192
NS = 8192
CHUNK = 128
NCHUNK = NS // CHUNK
NUM_WORKERS = 32
ROWS_PER_W = NQ // NUM_WORKERS


def _proj_body(x_ref, wt_ref, o_ref):
    o_ref[...] = jnp.dot(x_ref[...], wt_ref[...],
                         preferred_element_type=jnp.float32)


def _project(x, wt, br=1024):
    n = x.shape[0]
    d = x.shape[1]
    return pl.pallas_call(
        _proj_body,
        grid=(n // br,),
        in_specs=[pl.BlockSpec((br, d), lambda i: (i, 0)),
                  pl.BlockSpec((d, RPAD), lambda i: (0, 0))],
        out_specs=pl.BlockSpec((br, RPAD), lambda i: (i, 0)),
        out_shape=jax.ShapeDtypeStruct((n, RPAD), jnp.float32),
    )(x, wt)


def _score_body(scale, rq_ref, rkt_ref, mask_ref, s_ref, cm_ref):
    s = jnp.dot(rq_ref[...], rkt_ref[...],
                preferred_element_type=jnp.float32)
    s = s * scale + mask_ref[...]
    s_ref[...] = s
    br = s.shape[0]
    cm_ref[...] = jnp.max(s.reshape(br, NCHUNK, CHUNK), axis=2)


def _scores(rq, rkt, mask2d, scale, br=256):
    nq = rq.shape[0]
    grid = nq // br
    return pl.pallas_call(
        functools.partial(_score_body, scale),
        grid=(grid,),
        in_specs=[pl.BlockSpec((br, RPAD), lambda i: (i, 0)),
                  pl.BlockSpec((RPAD, NS), lambda i: (0, 0)),
                  pl.BlockSpec((1, NS), lambda i: (0, 0))],
        out_specs=[pl.BlockSpec((br, NS), lambda i: (i, 0)),
                   pl.BlockSpec((br, NCHUNK), lambda i: (i, 0))],
        out_shape=[jax.ShapeDtypeStruct((nq, NS), jnp.float32),
                   jax.ShapeDtypeStruct((nq, NCHUNK), jnp.float32)],
    )(rq, rkt, mask2d)


def _topk_body(rpw, scores_hbm, cmax_hbm, idx_hbm, val_hbm,
               row_a, row_b, row_c, row_d, m_all, idx_acc, val_acc,
               sem_a, sem_b, sem_c, sem_d):
    cc = lax.axis_index("c")
    ss = lax.axis_index("s")
    wid = ss * 2 + cc
    base = wid * rpw
    iota = lax.broadcasted_iota(jnp.int32, (16,), 0)
    lane0 = iota == 0
    NEG = jnp.float32(-jnp.inf)
    BIG = jnp.int32(1 << 30)

    def _put(ref, r, pos, value):
        plsc.store_scatter(ref,
                           [jnp.full((16,), r, jnp.int32),
                            jnp.full((16,), pos, jnp.int32)],
                           jnp.full((16,), value, ref.dtype), mask=lane0)

    def _shuf(x, s):
        return x.at[iota ^ s].get(mode="promise_in_bounds")

    def _lanemax(x):
        for sh in (8, 4, 2, 1):
            x = jnp.maximum(x, _shuf(x, sh))
        return x

    def _lanemin(x):
        for sh in (8, 4, 2, 1):
            x = jnp.minimum(x, _shuf(x, sh))
        return x

    pltpu.sync_copy(cmax_hbm.at[pl.ds(base, rpw)], m_all)
    pltpu.async_copy(scores_hbm.at[base], row_a, sem_a)
    pltpu.async_copy(scores_hbm.at[base + 1], row_b, sem_b)

    NEG_VEC = jnp.full((16,), NEG, jnp.float32)

    def step(i, m, r, row_v):
        mmv = jnp.maximum(jnp.maximum(m[0], m[1]),
                          jnp.maximum(m[2], m[3]))
        cmax = jnp.max(mmv)
        cand = None
        for j in range(4):
            fj = plsc.all_reduce_ffs(m[j] == cmax)
            cj = jnp.where(fj < 16, fj + (16 * j), BIG)
            cand = cj if cand is None else jnp.minimum(cand, cj)
        cid_v = cand
        start = cid_v[0] * CHUNK
        xs, pos = [], None
        for j in range(8):
            x = row_v[pl.ds(start + 16 * j, 16)]
            xs.append(x)
            fj = plsc.all_reduce_ffs(x == cmax)
            pj = jnp.where(fj < 16, (start + 16 * j) + fj, BIG)
            pos = pj if pos is None else jnp.minimum(pos, pj)
        p_v = pos
        nm = None
        for j in range(8):
            d = p_v - (start + 16 * j)
            xm = jnp.where(iota == d, NEG, xs[j])
            nm = xm if nm is None else jnp.maximum(nm, xm)
        newmax = jnp.max(nm)
        plsc.store_scatter(row_v, [p_v], NEG_VEC, mask=lane0)
        _put(idx_acc, r, i, p_v[0])
        _put(val_acc, r, i, cmax)
        cdiv = cid_v >> 4
        cmod = cid_v & 15
        return tuple(
            jnp.where((iota == cmod) & (cdiv == j), newmax, m[j])
            for j in range(4))

    def process_pair(r, row_x, row_y):
        def it_body(i, m):
            ma = step(i, m[:4], r, row_x)
            mb = step(i, m[4:], r + 1, row_y)
            return ma + mb

        m0 = tuple(m_all[r, pl.ds(16 * j, 16)] for j in range(4))
        m1 = tuple(m_all[r + 1, pl.ds(16 * j, 16)] for j in range(4))
        lax.fori_loop(0, TOPK, it_body, m0 + m1)

    def body4(q, carry):
        r0 = 4 * q
        pltpu.async_copy(scores_hbm.at[base + r0 + 2], row_c, sem_c)
        pltpu.async_copy(scores_hbm.at[base + r0 + 3], row_d, sem_d)
        pltpu.make_async_copy(scores_hbm.at[base + r0], row_a, sem_a).wait()
        pltpu.make_async_copy(scores_hbm.at[base + r0 + 1], row_b, sem_b).wait()
        process_pair(r0, row_a, row_b)

        @pl.when(q < rpw // 4 - 1)
        def _():
            pltpu.async_copy(scores_hbm.at[base + r0 + 4], row_a, sem_a)
            pltpu.async_copy(scores_hbm.at[base + r0 + 5], row_b, sem_b)

        pltpu.make_async_copy(scores_hbm.at[base + r0 + 2], row_c, sem_c).wait()
        pltpu.make_async_copy(scores_hbm.at[base + r0 + 3], row_d, sem_d).wait()
        process_pair(r0 + 2, row_c, row_d)
        return carry

    lax.fori_loop(0, rpw // 4, body4, 0)
    pltpu.sync_copy(idx_acc, idx_hbm.at[pl.ds(base, rpw)])
    pltpu.sync_copy(val_acc, val_hbm.at[pl.ds(base, rpw)])


def _topk(scores, cmax):
    nq = scores.shape[0]
    rpw = nq // NUM_WORKERS
    mesh = plsc.VectorSubcoreMesh(core_axis_name="c", subcore_axis_name="s")
    fn = pl.kernel(
        functools.partial(_topk_body, rpw),
        out_type=[jax.ShapeDtypeStruct((nq, TOPK), jnp.int32),
                  jax.ShapeDtypeStruct((nq, TOPK), jnp.float32)],
        mesh=mesh,
        compiler_params=pltpu.CompilerParams(needs_layout_passes=False),
        scratch_types=[pltpu.VMEM((NS,), jnp.float32),
                       pltpu.VMEM((NS,), jnp.float32),
                       pltpu.VMEM((NS,), jnp.float32),
                       pltpu.VMEM((NS,), jnp.float32),
                       pltpu.VMEM((rpw, NCHUNK), jnp.float32),
                       pltpu.VMEM((rpw, TOPK), jnp.int32),
                       pltpu.VMEM((rpw, TOPK), jnp.float32),
                       pltpu.SemaphoreType.DMA,
                       pltpu.SemaphoreType.DMA,
                       pltpu.SemaphoreType.DMA,
                       pltpu.SemaphoreType.DMA],
    )
    return fn(scores, cmax)


def kernel(query, slot_keys, reliability_mask, W_router):
    b, s, d = query.shape
    r = W_router.shape[0]
    scale = 1.0 / math.sqrt(r)
    q2 = query.reshape(b * s, d)
    wt = jnp.zeros((d, RPAD), jnp.float32).at[:, :r].set(W_router.T)
    rq = _project(q2, wt)
    rk = _project(slot_keys, wt)
    rkt = rk.T
    mask2d = reliability_mask.reshape(1, NS)
    ngroups = 2
    gsz = (b * s) // ngroups
    outs = []
    for g in range(ngroups):
        sc_g, cm_g = _scores(rq[g * gsz:(g + 1) * gsz], rkt, mask2d, scale)
        outs.append(_topk(sc_g, cm_g))
    idx = jnp.concatenate([o[0] for o in outs])
    val = jnp.concatenate([o[1] for o in outs])
    return idx.reshape(b, s, TOPK), val.reshape(b, s, TOPK)


# speedup vs baseline: 46.2963x; 1.0299x over previous
import functools
import math

import jax
import jax.numpy as jnp
from jax import lax
from jax.experimental import pallas as pl
from jax.experimental.pallas import tpu as pltpu
from jax.experimental.pallas import tpu_sc as plsc

TOPK = 32
RPAD = 64
NQ = 8192
NS = 8192
CHUNK = 128
NCHUNK = NS // CHUNK
NUM_WORKERS = 32
ROWS_PER_W = NQ // NUM_WORKERS


def _proj_body(x_ref, wt_ref, o_ref):
    o_ref[...] = jnp.dot(x_ref[...], wt_ref[...],
                         preferred_element_type=jnp.float32)


def _project(x, wt, br=1024):
    n = x.shape[0]
    d = x.shape[1]
    return pl.pallas_call(
        _proj_body,
        grid=(n // br,),
        in_specs=[pl.BlockSpec((br, d), lambda i: (i, 0)),
                  pl.BlockSpec((d, RPAD), lambda i: (0, 0))],
        out_specs=pl.BlockSpec((br, RPAD), lambda i: (i, 0)),
        out_shape=jax.ShapeDtypeStruct((n, RPAD), jnp.float32),
    )(x, wt)


def _score_body(scale, rq_ref, rkt_ref, mask_ref, s_ref, cm_ref):
    s = jnp.dot(rq_ref[...], rkt_ref[...],
                preferred_element_type=jnp.float32)
    s = s * scale + mask_ref[...]
    s_ref[...] = s
    br = s.shape[0]
    cm_ref[...] = jnp.max(s.reshape(br, NCHUNK, CHUNK), axis=2)


def _scores(rq, rkt, mask2d, scale, br=256):
    nq = rq.shape[0]
    grid = nq // br
    return pl.pallas_call(
        functools.partial(_score_body, scale),
        grid=(grid,),
        in_specs=[pl.BlockSpec((br, RPAD), lambda i: (i, 0)),
                  pl.BlockSpec((RPAD, NS), lambda i: (0, 0)),
                  pl.BlockSpec((1, NS), lambda i: (0, 0))],
        out_specs=[pl.BlockSpec((br, NS), lambda i: (i, 0)),
                   pl.BlockSpec((br, NCHUNK), lambda i: (i, 0))],
        out_shape=[jax.ShapeDtypeStruct((nq, NS), jnp.float32),
                   jax.ShapeDtypeStruct((nq, NCHUNK), jnp.float32)],
    )(rq, rkt, mask2d)


def _topk_body(rpw, scores_hbm, cmax_hbm, idx_hbm, val_hbm,
               row_a, row_b, row_c, row_d, m_all, idx_acc, val_acc,
               sem_a, sem_b, sem_c, sem_d):
    cc = lax.axis_index("c")
    ss = lax.axis_index("s")
    wid = ss * 2 + cc
    base = wid * rpw
    iota = lax.broadcasted_iota(jnp.int32, (16,), 0)
    lane0 = iota == 0
    NEG = jnp.float32(-jnp.inf)
    BIG = jnp.int32(1 << 30)

    def _put(ref, r, pos, value):
        plsc.store_scatter(ref,
                           [jnp.full((16,), r, jnp.int32),
                            jnp.full((16,), pos, jnp.int32)],
                           jnp.full((16,), value, ref.dtype), mask=lane0)

    def _shuf(x, s):
        return x.at[iota ^ s].get(mode="promise_in_bounds")

    def _lanemax(x):
        for sh in (8, 4, 2, 1):
            x = jnp.maximum(x, _shuf(x, sh))
        return x

    def _lanemin(x):
        for sh in (8, 4, 2, 1):
            x = jnp.minimum(x, _shuf(x, sh))
        return x

    pltpu.sync_copy(cmax_hbm.at[pl.ds(base, rpw)], m_all)
    pltpu.async_copy(scores_hbm.at[base], row_a, sem_a)
    pltpu.async_copy(scores_hbm.at[base + 1], row_b, sem_b)

    NEG_VEC = jnp.full((16,), NEG, jnp.float32)

    def step(i, m, r, row_v):
        mmv = jnp.maximum(jnp.maximum(m[0], m[1]),
                          jnp.maximum(m[2], m[3]))
        cmax = jnp.max(mmv)
        cand = None
        for j in range(4):
            fj = plsc.all_reduce_ffs(m[j] == cmax)
            cj = jnp.where(fj < 16, fj + (16 * j), BIG)
            cand = cj if cand is None else jnp.minimum(cand, cj)
        cid_v = cand
        start = cid_v[0] * CHUNK
        xs, pos = [], None
        for j in range(8):
            x = row_v[pl.ds(start + 16 * j, 16)]
            xs.append(x)
            fj = plsc.all_reduce_ffs(x == cmax)
            pj = jnp.where(fj < 16, (start + 16 * j) + fj, BIG)
            pos = pj if pos is None else jnp.minimum(pos, pj)
        p_v = pos
        nm = None
        for j in range(8):
            d = p_v - (start + 16 * j)
            xm = jnp.where(iota == d, NEG, xs[j])
            nm = xm if nm is None else jnp.maximum(nm, xm)
        newmax = jnp.max(nm)
        plsc.store_scatter(row_v, [p_v], NEG_VEC, mask=lane0)
        _put(idx_acc, r, i, p_v[0])
        _put(val_acc, r, i, cmax)
        cdiv = cid_v >> 4
        cmod = cid_v & 15
        return tuple(
            jnp.where((iota == cmod) & (cdiv == j), newmax, m[j])
            for j in range(4))

    def process_pair(r, row_x, row_y):
        def it_body(i, m):
            ma = step(i, m[:4], r, row_x)
            mb = step(i, m[4:], r + 1, row_y)
            return ma + mb

        m0 = tuple(m_all[r, pl.ds(16 * j, 16)] for j in range(4))
        m1 = tuple(m_all[r + 1, pl.ds(16 * j, 16)] for j in range(4))
        lax.fori_loop(0, TOPK, it_body, m0 + m1)

    def body4(q, carry):
        r0 = 4 * q
        pltpu.async_copy(scores_hbm.at[base + r0 + 2], row_c, sem_c)
        pltpu.async_copy(scores_hbm.at[base + r0 + 3], row_d, sem_d)
        pltpu.make_async_copy(scores_hbm.at[base + r0], row_a, sem_a).wait()
        pltpu.make_async_copy(scores_hbm.at[base + r0 + 1], row_b, sem_b).wait()
        process_pair(r0, row_a, row_b)

        @pl.when(q < rpw // 4 - 1)
        def _():
            pltpu.async_copy(scores_hbm.at[base + r0 + 4], row_a, sem_a)
            pltpu.async_copy(scores_hbm.at[base + r0 + 5], row_b, sem_b)

        pltpu.make_async_copy(scores_hbm.at[base + r0 + 2], row_c, sem_c).wait()
        pltpu.make_async_copy(scores_hbm.at[base + r0 + 3], row_d, sem_d).wait()
        process_pair(r0 + 2, row_c, row_d)
        return carry

    lax.fori_loop(0, rpw // 4, body4, 0)
    pltpu.sync_copy(idx_acc, idx_hbm.at[pl.ds(base, rpw)])
    pltpu.sync_copy(val_acc, val_hbm.at[pl.ds(base, rpw)])


def _topk(scores, cmax):
    nq = scores.shape[0]
    rpw = nq // NUM_WORKERS
    mesh = plsc.VectorSubcoreMesh(core_axis_name="c", subcore_axis_name="s")
    fn = pl.kernel(
        functools.partial(_topk_body, rpw),
        out_type=[jax.ShapeDtypeStruct((nq, TOPK), jnp.int32),
                  jax.ShapeDtypeStruct((nq, TOPK), jnp.float32)],
        mesh=mesh,
        compiler_params=pltpu.CompilerParams(needs_layout_passes=False),
        scratch_types=[pltpu.VMEM((NS,), jnp.float32),
                       pltpu.VMEM((NS,), jnp.float32),
                       pltpu.VMEM((NS,), jnp.float32),
                       pltpu.VMEM((NS,), jnp.float32),
                       pltpu.VMEM((rpw, NCHUNK), jnp.float32),
                       pltpu.VMEM((rpw, TOPK), jnp.int32),
                       pltpu.VMEM((rpw, TOPK), jnp.float32),
                       pltpu.SemaphoreType.DMA,
                       pltpu.SemaphoreType.DMA,
                       pltpu.SemaphoreType.DMA,
                       pltpu.SemaphoreType.DMA],
    )
    return fn(scores, cmax)


def kernel(query, slot_keys, reliability_mask, W_router):
    b, s, d = query.shape
    r = W_router.shape[0]
    scale = 1.0 / math.sqrt(r)
    q2 = query.reshape(b * s, d)
    wt = jnp.zeros((d, RPAD), jnp.float32).at[:, :r].set(W_router.T)
    rq = _project(q2, wt)
    rk = _project(slot_keys, wt)
    rkt = rk.T
    mask2d = reliability_mask.reshape(1, NS)
    ngroups = 4
    gsz = (b * s) // ngroups
    outs = []
    for g in range(ngroups):
        sc_g, cm_g = _scores(rq[g * gsz:(g + 1) * gsz], rkt, mask2d, scale)
        outs.append(_topk(sc_g, cm_g))
    idx = jnp.concatenate([o[0] for o in outs])
    val = jnp.concatenate([o[1] for o in outs])
    return idx.reshape(b, s, TOPK), val.reshape(b, s, TOPK)
